# baseline (speedup 1.0000x reference)
import jax
import jax.numpy as jnp
from jax.experimental import pallas as pl

NU = 50000
NI = 50000
NN = NU + NI
NE = 1600000
D = 32
TS = 16
NINT = 64
BATCH = 4096
TEMP = 0.2
KL_REG = 0.01
EMB_REG = 1e-05
INT_REG = 1e-05
SSL_REG = 0.1

RT = 1000
RG = 400
RB = 512


def _tail_kernel(e0, x1, x2, eps, intent, linWT, linb,
                 mean_o, gen_o, int_o, kl_o):
    mean = e0[...] + x1[...] + x2[...]
    mean_o[...] = mean
    std0 = jax.nn.softplus(mean[:, :TS])
    std = jnp.dot(std0, linWT[...], preferred_element_type=jnp.float32)
    std = std + linb[...] + 1e-8
    gen_o[...] = mean + eps[...] * std
    logits = jnp.dot(mean, intent[...], preferred_element_type=jnp.float32)
    p = jax.nn.softmax(logits, axis=1)
    int_o[...] = jnp.dot(p, intent[...].T, preferred_element_type=jnp.float32)
    kl = -0.5 * (1.0 + 2.0 * std - jnp.square(mean)
                 - jnp.square(jnp.exp(std)))
    kl = jnp.where(jnp.isinf(kl) | jnp.isnan(kl), 0.0, kl)

    @pl.when(pl.program_id(0) == 0)
    def _():
        kl_o[...] = jnp.zeros((1, 1), jnp.float32)

    kl_o[...] += jnp.sum(kl).reshape(1, 1)


def _tail(e0, x1, x2, eps, intent, linWT, linb):
    H = e0.shape[0]
    grid = H // RT
    row = lambda i: (i, 0)
    whole = lambda i: (0, 0)
    return pl.pallas_call(
        _tail_kernel,
        grid=(grid,),
        in_specs=[
            pl.BlockSpec((RT, D), row),
            pl.BlockSpec((RT, D), row),
            pl.BlockSpec((RT, D), row),
            pl.BlockSpec((RT, D), row),
            pl.BlockSpec((D, NINT), whole),
            pl.BlockSpec((TS, D), whole),
            pl.BlockSpec((1, D), whole),
        ],
        out_specs=[
            pl.BlockSpec((RT, D), row),
            pl.BlockSpec((RT, D), row),
            pl.BlockSpec((RT, D), row),
            pl.BlockSpec((1, 1), lambda i: (0, 0)),
        ],
        out_shape=[
            jax.ShapeDtypeStruct((H, D), jnp.float32),
            jax.ShapeDtypeStruct((H, D), jnp.float32),
            jax.ShapeDtypeStruct((H, D), jnp.float32),
            jax.ShapeDtypeStruct((1, 1), jnp.float32),
        ],
    )(e0, x1, x2, eps, intent, linWT, linb)


def _gather_kernel(idx, table, out):
    j = pl.program_id(0)

    @pl.when(j == 0)
    def _():
        out[...] = jnp.zeros_like(out)

    ids = idx[...]
    cols = j * RG + jax.lax.broadcasted_iota(jnp.int32, (1, RG), 1)
    oh = (ids == cols).astype(jnp.float32)
    out[...] += jnp.dot(oh, table[...], preferred_element_type=jnp.float32)


def _gather(idx2d, table):
    H, Dk = table.shape
    grid = H // RG
    return pl.pallas_call(
        _gather_kernel,
        grid=(grid,),
        in_specs=[
            pl.BlockSpec((BATCH, 1), lambda j: (0, 0)),
            pl.BlockSpec((RG, Dk), lambda j: (j, 0)),
        ],
        out_specs=pl.BlockSpec((BATCH, Dk), lambda j: (0, 0)),
        out_shape=jax.ShapeDtypeStruct((BATCH, Dk), jnp.float32),
    )(idx2d, table)


def _nrm(x):
    n = jnp.sqrt(jnp.sum(x * x, axis=1, keepdims=True))
    return x / jnp.maximum(n, 1e-12)


def _loss_kernel(ue, pe, ne, ug, ig, uif, iif, uemb, pemb, nemb,
                 uint_m, iint_m, bpr_o, emb_o, cl1_o, cl2_o, int_o):
    i = pl.program_id(0)

    i = pl.program_id(0)
    z = jnp.zeros((1, 1), jnp.float32)

    @pl.when(i == 0)
    def _():
        bpr_o[...] = z
        emb_o[...] = z
        cl1_o[...] = z
        cl2_o[...] = z
        int_o[...] = z

    pos_s = jnp.sum(ue[...] * pe[...], axis=1)
    neg_s = jnp.sum(ue[...] * ne[...], axis=1)
    bpr_o[...] += jnp.sum(jax.nn.softplus(neg_s - pos_s)).reshape(1, 1)
    emb_o[...] += (jnp.sum(jnp.square(uemb[...]))
                   + jnp.sum(jnp.square(pemb[...]))
                   + jnp.sum(jnp.square(nemb[...]))).reshape(1, 1)

    def cal(e1b, e2ref, o):
        e1 = _nrm(e1b)
        e2 = _nrm(e2ref[...])
        e2b = _nrm(e2ref[pl.ds(i * RB, RB), :])
        pos = jnp.exp(jnp.sum(e1 * e2b, axis=1) / TEMP)
        sim = jnp.dot(e1, e2.T, preferred_element_type=jnp.float32) / TEMP
        neg = jnp.sum(jnp.exp(sim), axis=1)
        o[...] += jnp.sum(-jnp.log(pos / (neg + 1e-8) + 1e-8)).reshape(1, 1)

    cal(ug[...], uif, cl1_o)
    cal(ig[...], iif, cl2_o)
    s = jnp.sum(jnp.square(uint_m[...])) + jnp.sum(jnp.square(iint_m[...]))
    int_o[...] += jnp.where(i == 0, s, 0.0).reshape(1, 1)


def _losses(ue, pe, ne, ug, ig, uif, iif, uemb, pemb, nemb, uint_m, iint_m):
    grid = BATCH // RB
    blk = lambda i: (i, 0)
    whole = lambda i: (0, 0)
    part = pl.BlockSpec((1, 1), lambda i: (0, 0))
    pshape = jax.ShapeDtypeStruct((1, 1), jnp.float32)
    return pl.pallas_call(
        _loss_kernel,
        grid=(grid,),
        in_specs=[
            pl.BlockSpec((RB, D), blk),
            pl.BlockSpec((RB, D), blk),
            pl.BlockSpec((RB, D), blk),
            pl.BlockSpec((RB, D), blk),
            pl.BlockSpec((RB, D), blk),
            pl.BlockSpec((BATCH, D), whole),
            pl.BlockSpec((BATCH, D), whole),
            pl.BlockSpec((RB, D), blk),
            pl.BlockSpec((RB, D), blk),
            pl.BlockSpec((RB, D), blk),
            pl.BlockSpec((D, NINT), whole),
            pl.BlockSpec((D, NINT), whole),
        ],
        out_specs=[part, part, part, part, part],
        out_shape=[pshape, pshape, pshape, pshape, pshape],
    )(ue, pe, ne, ug, ig, uif, iif, uemb, pemb, nemb, uint_m, iint_m)


def kernel(users, pos_items, neg_items, all_h_list, all_t_list, user_emb,
           item_emb, user_intent, item_intent, lin_W, lin_b, eps):
    h = all_h_list.astype(jnp.int32)
    t = all_t_list.astype(jnp.int32)
    users = users.astype(jnp.int32)
    pos_items = pos_items.astype(jnp.int32)
    neg_items = neg_items.astype(jnp.int32)

    deg = jax.ops.segment_sum(jnp.ones((NE,), jnp.float32), h,
                              num_segments=NN)
    deg = jnp.where(deg > 0, deg, 1.0)
    d_inv = deg ** -0.5
    g = d_inv[h] * d_inv[t]
    e0 = jnp.concatenate([user_emb, item_emb], axis=0)
    x1 = jax.ops.segment_sum(g[:, None] * e0[t], h, num_segments=NN)
    x2 = jax.ops.segment_sum(g[:, None] * x1[t], h, num_segments=NN)

    linWT = lin_W.T
    linb2 = lin_b.reshape(1, D)

    mean_u, gen_u, int_u, kl_u = _tail(e0[:NU], x1[:NU], x2[:NU], eps[:NU],
                                       user_intent, linWT, linb2)
    mean_i, gen_i, int_i, kl_i = _tail(e0[NU:], x1[NU:], x2[NU:], eps[NU:],
                                       item_intent, linWT, linb2)

    users2 = users.reshape(BATCH, 1)
    pos2 = pos_items.reshape(BATCH, 1)
    neg2 = neg_items.reshape(BATCH, 1)
    tab_u = jnp.concatenate([mean_u, gen_u, int_u, user_emb], axis=1)
    tab_p = jnp.concatenate([mean_i, gen_i, int_i, item_emb], axis=1)
    tab_n = jnp.concatenate([mean_i, item_emb], axis=1)
    gu = _gather(users2, tab_u)
    gp = _gather(pos2, tab_p)
    gn = _gather(neg2, tab_n)
    ue, ugen, uii, uemb = (gu[:, :D], gu[:, D:2 * D], gu[:, 2 * D:3 * D],
                           gu[:, 3 * D:])
    pe, igen, iii, pemb = (gp[:, :D], gp[:, D:2 * D], gp[:, 2 * D:3 * D],
                           gp[:, 3 * D:])
    ne, nemb = gn[:, :D], gn[:, D:]

    bpr_p, emb_p, cl1_p, cl2_p, int_p = _losses(
        ue, pe, ne, ugen, igen, uii, iii, uemb, pemb, nemb,
        user_intent, item_intent)

    bpr = jnp.sum(bpr_p) / BATCH
    kl_total = (jnp.sum(kl_u) + jnp.sum(kl_i)) / NN
    gen_loss = bpr + KL_REG * kl_total
    cl_loss = SSL_REG * (jnp.sum(cl1_p) / BATCH + jnp.sum(cl2_p) / BATCH)
    emb_loss = EMB_REG * jnp.sum(emb_p)
    int_loss = INT_REG * jnp.sum(int_p)
    return (gen_loss, cl_loss, emb_loss, int_loss)
